# baseline (speedup 1.0000x reference)
import functools

import jax
import jax.numpy as jnp
import numpy as np
from jax.experimental import pallas as pl

_DIM_HEAD = 32
_WINS = [8, 8, 4, 2]
_KS = [8, 6, 4, 2]


def _mm_body(x_ref, w_ref, g_ref, b_ref, *rest, act_out, pre, has_res):
    idx = 0
    if pre:
        pg_ref = rest[idx]; pb_ref = rest[idx + 1]; idx += 2
    if has_res:
        res_ref = rest[idx]; idx += 1
    o_ref = rest[idx]
    xv = x_ref[...]
    if pre:
        xv = jax.nn.gelu(xv * pg_ref[...] + pb_ref[...])
    acc = jnp.dot(xv, w_ref[...], preferred_element_type=jnp.float32)
    acc = acc * g_ref[...] + b_ref[...]
    if act_out:
        acc = jax.nn.gelu(acc)
    if has_res:
        acc = acc + res_ref[...]
    o_ref[...] = acc


def _mm(x, w, g, b, act_out=False, pre=None, res=None):
    M, K = x.shape
    N = w.shape[1]
    TM = min(M, 1024)
    assert M % TM == 0, (M, TM)
    inputs = [x, w, g.reshape(1, N), b.reshape(1, N)]
    in_specs = [
        pl.BlockSpec((TM, K), lambda i: (i, 0)),
        pl.BlockSpec((K, N), lambda i: (0, 0)),
        pl.BlockSpec((1, N), lambda i: (0, 0)),
        pl.BlockSpec((1, N), lambda i: (0, 0)),
    ]
    if pre is not None:
        pg, pb = pre
        inputs += [pg.reshape(1, K), pb.reshape(1, K)]
        in_specs += [pl.BlockSpec((1, K), lambda i: (0, 0)),
                     pl.BlockSpec((1, K), lambda i: (0, 0))]
    if res is not None:
        inputs.append(res)
        in_specs.append(pl.BlockSpec((TM, N), lambda i: (i, 0)))
    body = functools.partial(_mm_body, act_out=act_out, pre=pre is not None,
                             has_res=res is not None)
    return pl.pallas_call(
        body,
        grid=(M // TM,),
        in_specs=in_specs,
        out_specs=pl.BlockSpec((TM, N), lambda i: (i, 0)),
        out_shape=jax.ShapeDtypeStruct((M, N), jnp.float32),
    )(*inputs)


def _dw_body(x_ref, w_ref, g_ref, b_ref, o_ref, *, H, W):
    if H > 8:
        xv = x_ref[0].astype(jnp.bfloat16).astype(jnp.float32)
    else:
        xv = x_ref[0]
    wv = w_ref[...]
    acc = None
    for dy in range(3):
        for dx in range(3):
            term = xv[dy:dy + H, dx:dx + W, :] * wv[dy * 3 + dx]
            acc = term if acc is None else acc + term
    o_ref[0] = jax.nn.gelu(acc * g_ref[0] + b_ref[0])


def _dw3x3(x, wd, g, b):
    B, H, W, C = x.shape
    xp = jnp.pad(x, ((0, 0), (1, 1), (1, 1), (0, 0)))
    wd16 = jnp.pad(wd, ((0, 16 - 9), (0, 0)))
    body = functools.partial(_dw_body, H=H, W=W)
    return pl.pallas_call(
        body,
        grid=(B,),
        in_specs=[
            pl.BlockSpec((1, H + 2, W + 2, C), lambda i: (i, 0, 0, 0)),
            pl.BlockSpec((16, C), lambda i: (0, 0)),
            pl.BlockSpec((1, C), lambda i: (0, 0)),
            pl.BlockSpec((1, C), lambda i: (0, 0)),
        ],
        out_specs=pl.BlockSpec((1, H, W, C), lambda i: (i, 0, 0, 0)),
        out_shape=jax.ShapeDtypeStruct((B, H, W, C), jnp.float32),
    )(xp, wd16, g.reshape(1, C), b.reshape(1, C))


def _attn_body(q_ref, k_ref, v_ref, o_ref, *, nwin, T, topk, dh):
    qv = q_ref[0]
    kv = k_ref[0]
    vv = v_ref[0]
    L = nwin * T
    li = jax.lax.broadcasted_iota(jnp.int32, (nwin, L), 1) // T
    ni = jax.lax.broadcasted_iota(jnp.int32, (nwin, L), 0)
    P = jnp.where(li == ni, np.float32(1.0 / T), 0.0)
    qd = jnp.dot(P, qv, preferred_element_type=jnp.float32, precision=jax.lax.Precision.HIGHEST)
    kd = jnp.dot(P, kv, preferred_element_type=jnp.float32, precision=jax.lax.Precision.HIGHEST)
    s = jnp.dot(qd, kd.T, preferred_element_type=jnp.float32)
    sj = s[:, None, :]
    sm = s[:, :, None]
    ji = jax.lax.broadcasted_iota(jnp.int32, (nwin, nwin, nwin), 2)
    mi = jax.lax.broadcasted_iota(jnp.int32, (nwin, nwin, nwin), 1)
    gt = (sj > sm).astype(jnp.float32)
    tie = ((sj == sm) & (ji < mi)).astype(jnp.float32)
    rank = jnp.sum(gt + tie, axis=2)
    maskw = (rank < topk).astype(jnp.float32)
    li2 = jax.lax.broadcasted_iota(jnp.int32, (L, nwin), 0) // T
    ni2 = jax.lax.broadcasted_iota(jnp.int32, (L, nwin), 1)
    E = (li2 == ni2).astype(jnp.float32)
    bigmask = jnp.dot(jnp.dot(E, maskw, preferred_element_type=jnp.float32),
                      E.T, preferred_element_type=jnp.float32)
    sc = jnp.dot(qv, kv.T, preferred_element_type=jnp.float32) / np.float32(np.sqrt(dh))
    sc = jnp.where(bigmask > 0.5, sc, np.float32(-1e30))
    mx = jnp.max(sc, axis=1, keepdims=True)
    e = jnp.exp(sc - mx)
    p = e / jnp.sum(e, axis=1, keepdims=True)
    o_ref[0] = jnp.dot(p, vv, preferred_element_type=jnp.float32)


def _attn(q, k, v, nwin, T, topk):
    BH, L, dh = q.shape
    body = functools.partial(_attn_body, nwin=nwin, T=T, topk=topk, dh=dh)
    spec = pl.BlockSpec((1, L, dh), lambda i: (i, 0, 0))
    return pl.pallas_call(
        body,
        grid=(BH,),
        in_specs=[spec, spec, spec],
        out_specs=spec,
        out_shape=jax.ShapeDtypeStruct((BH, L, dh), jnp.float32),
    )(q, k, v)


def _head_body(x_ref, w_ref, b_ref, o_ref):
    xm = jnp.mean(x_ref[0], axis=0, keepdims=True)
    o_ref[0] = jnp.dot(xm, w_ref[...], preferred_element_type=jnp.float32) + b_ref[...]


def _head(x, w, b):
    B, L, C = x.shape
    N = w.shape[1]
    return pl.pallas_call(
        _head_body,
        grid=(B,),
        in_specs=[
            pl.BlockSpec((1, L, C), lambda i: (i, 0, 0)),
            pl.BlockSpec((C, N), lambda i: (0, 0)),
            pl.BlockSpec((1, N), lambda i: (0, 0)),
        ],
        out_specs=pl.BlockSpec((1, 1, N), lambda i: (i, 0, 0)),
        out_shape=jax.ShapeDtypeStruct((B, 1, N), jnp.float32),
    )(x, w, b.reshape(1, N))


def _im2col3x3(x, stride):
    B, H, W, C = x.shape
    if stride == 1:
        xp = jnp.pad(x, ((0, 0), (1, 1), (1, 1), (0, 0)))
        Ho, Wo = H, W
    else:
        xp = jnp.pad(x, ((0, 0), (0, 1), (0, 1), (0, 0)))
        Ho, Wo = H // stride, W // stride
    cols = []
    for dy in range(3):
        for dx in range(3):
            sl = jax.lax.slice(
                xp, (0, dy, dx, 0),
                (B, dy + (Ho - 1) * stride + 1, dx + (Wo - 1) * stride + 1, C),
                (1, stride, stride, 1))
            cols.append(sl)
    return jnp.concatenate(cols, axis=-1), Ho, Wo


def _w3x3(w):
    return w.transpose(2, 3, 1, 0).reshape(-1, w.shape[0])


def _w1x1(w):
    return w[:, :, 0, 0].T


def _to_heads(t, heads, w):
    B, H, W, D = t.shape
    nh, nw = H // w, W // w
    dh = D // heads
    t = t.reshape(B, nh, w, nw, w, heads, dh)
    t = t.transpose(0, 5, 1, 3, 2, 4, 6)
    return t.reshape(B * heads, nh * nw * w * w, dh)


def _from_heads(t, B, heads, H, W, w):
    dh = t.shape[-1]
    nh, nw = H // w, W // w
    t = t.reshape(B, heads, nh, nw, w, w, dh)
    t = t.transpose(0, 2, 4, 3, 5, 1, 6)
    return t.reshape(B, H, W, heads * dh)


def _conv3x3_bn_gelu(x, w, g, b, stride):
    B = x.shape[0]
    cols, Ho, Wo = _im2col3x3(x, stride)
    K = cols.shape[-1]
    wm = _w3x3(w)
    if K % 128 != 0:
        Kp = ((K + 127) // 128) * 128
        cols = jnp.pad(cols, ((0, 0), (0, 0), (0, 0), (0, Kp - K)))
        wm = jnp.pad(wm, ((0, Kp - K), (0, 0)))
        K = Kp
    out = _mm(cols.reshape(B * Ho * Wo, K), wm, g, b, act_out=True)
    return out.reshape(B, Ho, Wo, -1)


def _mbconv(x, p):
    B, H, W, C = x.shape
    mid = p['wd'].shape[0]
    M = B * H * W
    h1 = _mm(x.reshape(M, C), _w1x1(p['w1']), p['g1'], p['b1'], act_out=True)
    h1 = h1.reshape(B, H, W, mid)
    wd = p['wd'][:, 0, :, :].transpose(1, 2, 0).reshape(9, mid)
    h2 = _dw3x3(h1, wd, p['g2'], p['b2'])
    out = _mm(h2.reshape(M, mid), _w1x1(p['w2']), p['g3'], p['b3'],
              res=x.reshape(M, C))
    return out.reshape(B, H, W, C)


def _attn_layer(x, p, win, topk):
    B, H, W, D = x.shape
    heads = D // _DIM_HEAD
    M = B * H * W
    ones = jnp.ones((3 * D,), jnp.float32)
    zeros = jnp.zeros((3 * D,), jnp.float32)
    qkv = _mm(x.reshape(M, D), _w1x1(p['qkv_w']), ones, zeros,
              pre=(p['qkv_g'], p['qkv_b']))
    qkv = qkv.reshape(B, H, W, 3 * D)
    q, k, v = qkv[..., :D], qkv[..., D:2 * D], qkv[..., 2 * D:]
    nwin = (H // win) * (W // win)
    T = win * win
    qh = _to_heads(q, heads, win)
    kh = _to_heads(k, heads, win)
    vh = _to_heads(v, heads, win)
    oh = _attn(qh, kh, vh, nwin, T, topk)
    msg = _from_heads(oh, B, heads, H, W, win)
    x = _mm(msg.reshape(M, D), _w1x1(p['merge_w']),
            jnp.ones((D,), jnp.float32), p['merge_b'],
            res=x.reshape(M, D)).reshape(B, H, W, D)
    return _mbconv(x, p['mlp'])


def kernel(x, params):
    x = x.transpose(0, 2, 3, 1)
    x = _conv3x3_bn_gelu(x, params['stem1_w'], params['stem1_g'],
                         params['stem1_b'], 2)
    x = _conv3x3_bn_gelu(x, params['stem2_w'], params['stem2_g'],
                         params['stem2_b'], 2)
    for i, st in enumerate(params['stages']):
        if i > 0:
            x = _conv3x3_bn_gelu(x, st['down_w'], st['down_g'],
                                 st['down_b'], 2)
        for cb in st['conv_blocks']:
            x = _mbconv(x, cb['conv'])
            x = _mbconv(x, cb['mlp'])
        for ab in st['attn_blocks']:
            x = _attn_layer(x, ab, _WINS[i], _KS[i])
    B, H, W, C = x.shape
    wh = params['head_w'].T
    N = wh.shape[1]
    Np = 128
    wh = jnp.pad(wh, ((0, 0), (0, Np - N)))
    bh = jnp.pad(params['head_b'], (0, Np - N))
    out = _head(x.reshape(B, H * W, C), wh, bh)
    return out.reshape(B, Np)[:, :N]
